# baseline (speedup 1.0000x reference)
import functools

import jax
import jax.numpy as jnp
from jax import lax
from jax.experimental import pallas as pl
from jax.experimental.pallas import tpu as pltpu
from jax.experimental.pallas import tpu_sc as plsc

B = 2
F = 128
N_LOW = 40962
N_HIGH = 163842

VB1 = 512
G1 = 81
N_LOW_PAD = G1 * VB1

NW = 32
K = 256
NCH = 21
C = K * NCH
N_HIGH_PAD = NW * C

VB3 = 512
G3 = 321


def _mm_body(x_ref, w_ref, y_ref):
    w = w_ref[...]
    for bb in range(B):
        xb = x_ref[bb]
        yb = lax.dot_general(xb, w, (((0,), (1,)), ((), ())),
                             preferred_element_type=jnp.float32)
        y_ref[bb] = yb * 0.5


_info = plsc.get_sparse_core_info()
_NC = _info.num_cores
_NS = _info.num_subcores


@functools.partial(
    pl.kernel,
    mesh=plsc.VectorSubcoreMesh(core_axis_name="c", subcore_axis_name="s"),
    compiler_params=pltpu.CompilerParams(use_tc_tiling_on_sc=False),
    out_type=jax.ShapeDtypeStruct((B, N_HIGH_PAD, F), jnp.float32),
    scratch_types=[
        pltpu.VMEM((NCH, K), jnp.int32),
        pltpu.VMEM((NCH, K), jnp.int32),
        pltpu.VMEM((K, F), jnp.float32),
        pltpu.SemaphoreType.DMA,
    ],
)
def _sc_gather(y_hbm, idx0_hbm, idx1_hbm, h_hbm, idx0_v, idx1_v, out_v, sem):
    wid = lax.axis_index("s") * _NC + lax.axis_index("c")
    pltpu.sync_copy(idx0_hbm.at[wid], idx0_v)
    pltpu.sync_copy(idx1_hbm.at[wid], idx1_v)
    base = wid * C

    for bb in range(B):
        def chunk(j, carry):
            pltpu.async_copy(y_hbm.at[bb].at[idx0_v.at[j]], out_v, sem).wait()
            pltpu.async_copy(
                y_hbm.at[bb].at[idx1_v.at[j]], out_v, sem, add=True).wait()
            pltpu.sync_copy(out_v, h_hbm.at[bb].at[pl.ds(base + j * K, K)])
            return carry

        lax.fori_loop(0, NCH, chunk, 0)


def _tr_body(h_ref, bias_ref, o_ref):
    hb = h_ref[0]
    o_ref[0] = jnp.transpose(hb) + bias_ref[...]


def kernel(x, up_neigh_indices, W, b):
    y = pl.pallas_call(
        _mm_body,
        grid=(G1,),
        in_specs=[
            pl.BlockSpec((B, F, VB1), lambda j: (0, 0, j)),
            pl.BlockSpec((F, F), lambda j: (0, 0)),
        ],
        out_specs=pl.BlockSpec((B, VB1, F), lambda j: (0, j, 0)),
        out_shape=jax.ShapeDtypeStruct((B, N_LOW_PAD, F), jnp.float32),
    )(x, W)

    idx = jnp.concatenate(
        [up_neigh_indices,
         jnp.zeros((N_HIGH_PAD - N_HIGH, 2), jnp.int32)], axis=0)
    idx0 = idx[:, 0].reshape(NW, NCH, K)
    idx1 = idx[:, 1].reshape(NW, NCH, K)

    h = _sc_gather(y, idx0, idx1)

    bias_tile = jnp.tile(b[:, None], (1, VB3))
    out = pl.pallas_call(
        _tr_body,
        grid=(B, G3),
        in_specs=[
            pl.BlockSpec((1, VB3, F), lambda bb, j: (bb, j, 0)),
            pl.BlockSpec((F, VB3), lambda bb, j: (0, 0)),
        ],
        out_specs=pl.BlockSpec((1, F, VB3), lambda bb, j: (bb, 0, j)),
        out_shape=jax.ShapeDtypeStruct((B, F, N_HIGH), jnp.float32),
    )(h, bias_tile)
    return out


# speedup vs baseline: 1.4942x; 1.4942x over previous
import functools

import jax
import jax.numpy as jnp
from jax import lax
from jax.experimental import pallas as pl
from jax.experimental.pallas import tpu as pltpu
from jax.experimental.pallas import tpu_sc as plsc

B = 2
F = 128
N_LOW = 40962
N_HIGH = 163842

VB1 = 512
G1 = 81
N_LOW_PAD = G1 * VB1

NW = 32
K = 384
NCH = 14
C = K * NCH
N_HIGH_PAD = NW * C

VB3 = 512
G3 = 321


def _mm_body(x_ref, w_ref, y_ref):
    w = w_ref[...]
    for bb in range(B):
        xb = x_ref[bb]
        yb = lax.dot_general(xb, w, (((0,), (1,)), ((), ())),
                             preferred_element_type=jnp.float32)
        y_ref[:, bb, :] = yb * 0.5


_info = plsc.get_sparse_core_info()
_NC = _info.num_cores
_NS = _info.num_subcores


@functools.partial(
    pl.kernel,
    mesh=plsc.VectorSubcoreMesh(core_axis_name="c", subcore_axis_name="s"),
    compiler_params=pltpu.CompilerParams(use_tc_tiling_on_sc=False),
    out_type=jax.ShapeDtypeStruct((N_HIGH_PAD, B, F), jnp.float32),
    scratch_types=[
        pltpu.VMEM((NCH, K), jnp.int32),
        pltpu.VMEM((NCH, K), jnp.int32),
        pltpu.VMEM((K, B, F), jnp.float32),
        pltpu.SemaphoreType.DMA,
    ],
)
def _sc_gather(y_hbm, idx0_hbm, idx1_hbm, h_hbm, idx0_v, idx1_v, out_v, sem):
    wid = lax.axis_index("s") * _NC + lax.axis_index("c")
    pltpu.sync_copy(idx0_hbm.at[wid], idx0_v)
    pltpu.sync_copy(idx1_hbm.at[wid], idx1_v)
    base = wid * C

    def chunk(j, carry):
        pltpu.async_copy(y_hbm.at[idx0_v.at[j]], out_v, sem).wait()
        pltpu.async_copy(y_hbm.at[idx1_v.at[j]], out_v, sem, add=True).wait()
        pltpu.sync_copy(out_v, h_hbm.at[pl.ds(base + j * K, K)])
        return carry

    lax.fori_loop(0, NCH, chunk, 0)


def _tr_body(h_ref, bias_ref, o_ref):
    for bb in range(B):
        o_ref[bb] = jnp.transpose(h_ref[:, bb, :]) + bias_ref[...]


def kernel(x, up_neigh_indices, W, b):
    y = pl.pallas_call(
        _mm_body,
        grid=(G1,),
        in_specs=[
            pl.BlockSpec((B, F, VB1), lambda j: (0, 0, j)),
            pl.BlockSpec((F, F), lambda j: (0, 0)),
        ],
        out_specs=pl.BlockSpec((VB1, B, F), lambda j: (j, 0, 0)),
        out_shape=jax.ShapeDtypeStruct((N_LOW_PAD, B, F), jnp.float32),
    )(x, W)

    idx = jnp.concatenate(
        [up_neigh_indices,
         jnp.zeros((N_HIGH_PAD - N_HIGH, 2), jnp.int32)], axis=0)
    idx0 = idx[:, 0].reshape(NW, NCH, K)
    idx1 = idx[:, 1].reshape(NW, NCH, K)

    h = _sc_gather(y, idx0, idx1)

    bias_tile = jnp.tile(b[:, None], (1, VB3))
    out = pl.pallas_call(
        _tr_body,
        grid=(G3,),
        in_specs=[
            pl.BlockSpec((VB3, B, F), lambda j: (j, 0, 0)),
            pl.BlockSpec((F, VB3), lambda j: (0, 0)),
        ],
        out_specs=pl.BlockSpec((B, F, VB3), lambda j: (0, 0, j)),
        out_shape=jax.ShapeDtypeStruct((B, F, N_HIGH), jnp.float32),
    )(h, bias_tile)
    return out
